# baseline (speedup 1.0000x reference)
import functools

import jax
import jax.numpy as jnp
from jax import lax
from jax.experimental import pallas as pl
from jax.experimental.pallas import tpu as pltpu
from jax.experimental.pallas import tpu_sc as plsc

_N = 10000
_E = 320000
_D = 128
_NT = 17
_ET = 9

_NC = 2
_NS = 16
_NW = _NC * _NS

_NACC = 10240
_ZCH = 640
_WCH = 632
_NOUT = _NS * _WCH
_TRASH = 10224

_EPW_MSG = 10240
_EPW_EMB = 2048
_IDXR = 16


@functools.lru_cache(maxsize=None)
def _make_sc_gather_scatter_add(rows_per_worker):
    r_per_w = rows_per_worker // 128

    mesh = plsc.VectorSubcoreMesh(core_axis_name="c", subcore_axis_name="s")

    @functools.partial(
        pl.kernel,
        out_type=jax.ShapeDtypeStruct((2 * _NOUT, _D), jnp.float32),
        mesh=mesh,
        scratch_types=[
            pltpu.VMEM((_IDXR, 128), jnp.int32),
            pltpu.VMEM((_IDXR, 128), jnp.int32),
            pltpu.VMEM((128, _D), jnp.float32),
            pltpu.VMEM((128, _D), jnp.float32),
            pltpu.VMEM_SHARED((_NACC, _D), jnp.float32),
            pltpu.SemaphoreType.DMA,
            pltpu.SemaphoreType.DMA,
        ],
    )
    def k(tab_hbm, v_hbm, d_hbm, zero_hbm, out_hbm, vall, dall, rows_a, rows_b,
          accum, sem_a, sem_b):
        c = lax.axis_index("c")
        s = lax.axis_index("s")
        wid = c * _NS + s
        base = wid * r_per_w

        pltpu.sync_copy(zero_hbm, accum.at[pl.ds(s * _ZCH, _ZCH)])
        plsc.subcore_barrier()

        def gather(buf, sem, j):
            return pltpu.make_async_copy(tab_hbm.at[vall.at[j]], buf, sem)

        @pl.loop(0, r_per_w, step=_IDXR)
        def _(r0):
            pltpu.sync_copy(v_hbm.at[pl.ds(base + r0, _IDXR)], vall)
            pltpu.sync_copy(d_hbm.at[pl.ds(base + r0, _IDXR)], dall)

            gather(rows_a, sem_a, 0).start()

            @pl.loop(0, _IDXR, step=2)
            def _(t):
                gather(rows_b, sem_b, t + 1).start()
                gather(rows_a, sem_a, t).wait()
                pltpu.sync_copy(rows_a, accum.at[dall.at[t]], add=True)

                @pl.when(t + 2 < _IDXR)
                def _():
                    gather(rows_a, sem_a, t + 2).start()

                gather(rows_b, sem_b, t + 1).wait()
                pltpu.sync_copy(rows_b, accum.at[dall.at[t + 1]], add=True)

        plsc.subcore_barrier()
        pltpu.sync_copy(
            accum.at[pl.ds(s * _WCH, _WCH)],
            out_hbm.at[pl.ds(c * _NOUT + s * _WCH, _WCH)],
        )

    return k


@functools.lru_cache(maxsize=None)
def _make_sc_gather(rows_per_worker, n_out_rows):
    r_per_w = rows_per_worker // 128

    mesh = plsc.VectorSubcoreMesh(core_axis_name="c", subcore_axis_name="s")

    @functools.partial(
        pl.kernel,
        out_type=jax.ShapeDtypeStruct((n_out_rows, _D), jnp.float32),
        mesh=mesh,
        scratch_types=[
            pltpu.VMEM((_IDXR, 128), jnp.int32),
            pltpu.VMEM((128, _D), jnp.float32),
            pltpu.VMEM((128, _D), jnp.float32),
            pltpu.SemaphoreType.DMA,
            pltpu.SemaphoreType.DMA,
        ],
    )
    def k(tab_hbm, v_hbm, out_hbm, vall, rows_a, rows_b, sem_a, sem_b):
        c = lax.axis_index("c")
        s = lax.axis_index("s")
        wid = c * _NS + s
        base = wid * r_per_w

        def gather(buf, sem, j):
            return pltpu.make_async_copy(tab_hbm.at[vall.at[j]], buf, sem)

        def put(buf, r):
            pltpu.sync_copy(buf, out_hbm.at[pl.ds((base + r) * 128, 128)])

        @pl.loop(0, r_per_w, step=_IDXR)
        def _(r0):
            pltpu.sync_copy(v_hbm.at[pl.ds(base + r0, _IDXR)], vall)
            gather(rows_a, sem_a, 0).start()

            @pl.loop(0, _IDXR, step=2)
            def _(t):
                gather(rows_b, sem_b, t + 1).start()
                gather(rows_a, sem_a, t).wait()
                put(rows_a, r0 + t)

                @pl.when(t + 2 < _IDXR)
                def _():
                    gather(rows_a, sem_a, t + 2).start()

                gather(rows_b, sem_b, t + 1).wait()
                put(rows_b, r0 + t + 1)

    return k


_NB = 1000
_NGRID = _N // _NB


def _combine_body(srows, xp, ntc, ntt, ett, xf_ref, z_ref):
    cnt = jnp.sum((xp[...] != 0).astype(jnp.float32), axis=1, keepdims=True)
    recip = 1.0 / jnp.maximum(cnt, 1.0)
    sr = srows[...]
    ssum = sr[:, 0 * _D:1 * _D]
    for p in range(1, 5):
        ssum = ssum + sr[:, p * _D:(p + 1) * _D]
    sub = ssum * recip
    oh = (lax.broadcasted_iota(jnp.int32, (_NB, 32), 1) == ntc[...]).astype(
        jnp.float32)
    xf = sub + jnp.dot(oh, ntt[...], preferred_element_type=jnp.float32,
                       precision=lax.Precision.HIGHEST)
    xf_ref[...] = xf
    for t in range(_ET):
        z_ref[t] = jnp.maximum(xf + ett[t], 0.0)


_combine = pl.pallas_call(
    _combine_body,
    grid=(_NGRID,),
    in_specs=[
        pl.BlockSpec((_NB, 5 * _D), lambda i: (i, 0)),
        pl.BlockSpec((_NB, 8), lambda i: (i, 0)),
        pl.BlockSpec((_NB, 1), lambda i: (i, 0)),
        pl.BlockSpec((32, _D), lambda i: (0, 0)),
        pl.BlockSpec((16, _D), lambda i: (0, 0)),
    ],
    out_specs=[
        pl.BlockSpec((_NB, _D), lambda i: (i, 0)),
        pl.BlockSpec((_ET, _NB, _D), lambda i: (0, i, 0)),
    ],
    out_shape=[
        jax.ShapeDtypeStruct((_N, _D), jnp.float32),
        jax.ShapeDtypeStruct((_ET, _N, _D), jnp.float32),
    ],
)


def _mlp1_body(xf, a0, a1, w1, b1, h_ref, s1_ref, s2_ref):
    i = pl.program_id(0)
    h0 = xf[...] + a0[...] + a1[...]
    h = jnp.dot(h0, w1[...], preferred_element_type=jnp.float32) + b1[...]
    h_ref[...] = h
    ps = jnp.sum(h, axis=0, keepdims=True)
    pq = jnp.sum(h * h, axis=0, keepdims=True)

    @pl.when(i == 0)
    def _():
        s1_ref[...] = ps
        s2_ref[...] = pq

    @pl.when(i > 0)
    def _():
        s1_ref[...] += ps
        s2_ref[...] += pq


_mlp1 = pl.pallas_call(
    _mlp1_body,
    grid=(_NGRID,),
    in_specs=[
        pl.BlockSpec((_NB, _D), lambda i: (i, 0)),
        pl.BlockSpec((_NB, _D), lambda i: (i, 0)),
        pl.BlockSpec((_NB, _D), lambda i: (i, 0)),
        pl.BlockSpec((_D, 2 * _D), lambda i: (0, 0)),
        pl.BlockSpec((1, 2 * _D), lambda i: (0, 0)),
    ],
    out_specs=[
        pl.BlockSpec((_NB, 2 * _D), lambda i: (i, 0)),
        pl.BlockSpec((1, 2 * _D), lambda i: (0, 0)),
        pl.BlockSpec((1, 2 * _D), lambda i: (0, 0)),
    ],
    out_shape=[
        jax.ShapeDtypeStruct((_N, 2 * _D), jnp.float32),
        jax.ShapeDtypeStruct((1, 2 * _D), jnp.float32),
        jax.ShapeDtypeStruct((1, 2 * _D), jnp.float32),
    ],
)


def _mlp2_core(h, s1, s2, g, be, w2, b2):
    mu = s1[...] * (1.0 / _N)
    var = s2[...] * (1.0 / _N) - mu * mu
    inv = lax.rsqrt(var + 1e-5)
    hn = (h[...] - mu) * (inv * g[...]) + be[...]
    r = jnp.maximum(hn, 0.0)
    return jnp.dot(r, w2[...], preferred_element_type=jnp.float32) + b2[...]


def _mlp2z_body(h, s1, s2, g, be, w2, b2, ett, xf_ref, z_ref):
    o = _mlp2_core(h, s1, s2, g, be, w2, b2)
    xf_ref[...] = o
    for t in range(_ET):
        z_ref[t] = jnp.maximum(o + ett[t], 0.0)


def _mlp2_body(h, s1, s2, g, be, w2, b2, xf_ref):
    xf_ref[...] = _mlp2_core(h, s1, s2, g, be, w2, b2)


_mlp2_common_specs = [
    pl.BlockSpec((_NB, 2 * _D), lambda i: (i, 0)),
    pl.BlockSpec((1, 2 * _D), lambda i: (0, 0)),
    pl.BlockSpec((1, 2 * _D), lambda i: (0, 0)),
    pl.BlockSpec((1, 2 * _D), lambda i: (0, 0)),
    pl.BlockSpec((1, 2 * _D), lambda i: (0, 0)),
    pl.BlockSpec((2 * _D, _D), lambda i: (0, 0)),
    pl.BlockSpec((1, _D), lambda i: (0, 0)),
]

_mlp2z = pl.pallas_call(
    _mlp2z_body,
    grid=(_NGRID,),
    in_specs=_mlp2_common_specs + [pl.BlockSpec((16, _D), lambda i: (0, 0))],
    out_specs=[
        pl.BlockSpec((_NB, _D), lambda i: (i, 0)),
        pl.BlockSpec((_ET, _NB, _D), lambda i: (0, i, 0)),
    ],
    out_shape=[
        jax.ShapeDtypeStruct((_N, _D), jnp.float32),
        jax.ShapeDtypeStruct((_ET, _N, _D), jnp.float32),
    ],
)

_mlp2 = pl.pallas_call(
    _mlp2_body,
    grid=(_NGRID,),
    in_specs=_mlp2_common_specs,
    out_specs=[pl.BlockSpec((_NB, _D), lambda i: (i, 0))],
    out_shape=[jax.ShapeDtypeStruct((_N, _D), jnp.float32)],
)


def _pad_ids(ids, total, fill):
    ids = ids.astype(jnp.int32)
    return jnp.concatenate(
        [ids, jnp.full((total - ids.shape[0],), fill, jnp.int32)]
    ).reshape(total // 128, 128)


def kernel(x, node_type, edge_type, edge_index, st_table, nt_table, et_table,
           W1_0, b1_0, g_0, be_0, W2_0, b2_0,
           W1_1, b1_1, g_1, be_1, W2_1, b2_1,
           W1_2, b1_2, g_2, be_2, W2_2, b2_2):
    src = edge_index[0].astype(jnp.int32)
    dst = edge_index[1].astype(jnp.int32)
    et = edge_type.astype(jnp.int32)

    e_pad = _NW * _EPW_MSG
    v2 = _pad_ids(et * _N + src, e_pad, 0)
    d2 = _pad_ids(dst, e_pad, _TRASH)

    n_ids = _NW * _EPW_EMB
    xe = _pad_ids(x.reshape(-1), n_ids, 0)

    zero_blk = jnp.zeros((_ZCH, _D), jnp.float32)
    st_z = st_table.at[0].set(0.0)

    srows = _make_sc_gather(_EPW_EMB, n_ids)(st_z, xe)
    srows = srows[:_N * x.shape[1]].reshape(_N, x.shape[1] * _D)

    xpad = jnp.pad(x.astype(jnp.int32), ((0, 0), (0, 8 - x.shape[1])))
    ntc = node_type.astype(jnp.int32).reshape(_N, 1)
    ntt = jnp.pad(nt_table, ((0, 32 - _NT), (0, 0)))
    ett = jnp.pad(et_table, ((0, 16 - _ET), (0, 0)))
    xf, z = _combine(srows, xpad, ntc, ntt, ett)

    layers = [
        (W1_0, b1_0, g_0, be_0, W2_0, b2_0),
        (W1_1, b1_1, g_1, be_1, W2_1, b2_1),
        (W1_2, b1_2, g_2, be_2, W2_2, b2_2),
    ]
    for l, (w1, b1, g, be, w2, b2) in enumerate(layers):
        aggr = _make_sc_gather_scatter_add(_EPW_MSG)(
            z.reshape(_ET * _N, _D), v2, d2, zero_blk)
        h, s1, s2 = _mlp1(xf, aggr[:_N], aggr[_NOUT:_NOUT + _N], w1,
                          b1.reshape(1, -1))
        args = (h, s1, s2, g.reshape(1, -1), be.reshape(1, -1), w2,
                b2.reshape(1, -1))
        if l < 2:
            xf, z = _mlp2z(*args, ett)
        else:
            (xf,) = _mlp2(*args)
    return xf


# speedup vs baseline: 3.9768x; 3.9768x over previous
import functools

import jax
import jax.numpy as jnp
from jax import lax
from jax.experimental import pallas as pl
from jax.experimental.pallas import tpu as pltpu
from jax.experimental.pallas import tpu_sc as plsc

_N = 10000
_E = 320000
_D = 128
_NT = 17
_ET = 9

_NC = 2
_NS = 16
_NW = _NC * _NS

_NACC = 10240
_ZCH = 640
_WCH = 632
_NOUT = _NS * _WCH


_EPW_MSG = 10240
_EPW_EMB = 2048
_IDXR = 16


@functools.lru_cache(maxsize=None)
def _make_sc_gather_scatter_add(rows_per_worker):
    r_per_w = rows_per_worker // 128

    mesh = plsc.VectorSubcoreMesh(core_axis_name="c", subcore_axis_name="s")

    @functools.partial(
        pl.kernel,
        out_type=jax.ShapeDtypeStruct((2 * _NOUT, _D), jnp.float32),
        mesh=mesh,
        scratch_types=[
            pltpu.VMEM((_IDXR, 128), jnp.int32),
            pltpu.VMEM((_IDXR, 128), jnp.int32),
            pltpu.VMEM((128, _D), jnp.float32),
            pltpu.VMEM((128, _D), jnp.float32),
            pltpu.VMEM_SHARED((_NACC, _D), jnp.float32),
            pltpu.SemaphoreType.DMA,
            pltpu.SemaphoreType.DMA,
        ],
    )
    def k(tab_hbm, v_hbm, d_hbm, zero_hbm, out_hbm, vall, dall, rows_a, rows_b,
          accum, sem_a, sem_b):
        c = lax.axis_index("c")
        s = lax.axis_index("s")
        wid = c * _NS + s
        base = wid * r_per_w

        pltpu.sync_copy(zero_hbm, accum.at[pl.ds(s * _ZCH, _ZCH)])
        plsc.subcore_barrier()

        def gather(buf, sem, j):
            return pltpu.make_async_copy(tab_hbm.at[vall.at[j]], buf, sem)

        @pl.loop(0, r_per_w, step=_IDXR)
        def _(r0):
            pltpu.sync_copy(v_hbm.at[pl.ds(base + r0, _IDXR)], vall)
            pltpu.sync_copy(d_hbm.at[pl.ds(base + r0, _IDXR)], dall)

            gather(rows_a, sem_a, 0).start()

            @pl.loop(0, _IDXR, step=2)
            def _(t):
                gather(rows_b, sem_b, t + 1).start()
                gather(rows_a, sem_a, t).wait()
                pltpu.sync_copy(rows_a, accum.at[dall.at[t]], add=True)

                @pl.when(t + 2 < _IDXR)
                def _():
                    gather(rows_a, sem_a, t + 2).start()

                gather(rows_b, sem_b, t + 1).wait()
                pltpu.sync_copy(rows_b, accum.at[dall.at[t + 1]], add=True)

        plsc.subcore_barrier()
        pltpu.sync_copy(
            accum.at[pl.ds(s * _WCH, _WCH)],
            out_hbm.at[pl.ds(c * _NOUT + s * _WCH, _WCH)],
        )

    return k


@functools.lru_cache(maxsize=None)
def _make_sc_gather(rows_per_worker, n_out_rows):
    r_per_w = rows_per_worker // 128

    mesh = plsc.VectorSubcoreMesh(core_axis_name="c", subcore_axis_name="s")

    @functools.partial(
        pl.kernel,
        out_type=jax.ShapeDtypeStruct((n_out_rows, _D), jnp.float32),
        mesh=mesh,
        scratch_types=[
            pltpu.VMEM((_IDXR, 128), jnp.int32),
            pltpu.VMEM((128, _D), jnp.float32),
            pltpu.VMEM((128, _D), jnp.float32),
            pltpu.SemaphoreType.DMA,
            pltpu.SemaphoreType.DMA,
        ],
    )
    def k(tab_hbm, v_hbm, out_hbm, vall, rows_a, rows_b, sem_a, sem_b):
        c = lax.axis_index("c")
        s = lax.axis_index("s")
        wid = c * _NS + s
        base = wid * r_per_w

        def gather(buf, sem, j):
            return pltpu.make_async_copy(tab_hbm.at[vall.at[j]], buf, sem)

        def put(buf, r):
            pltpu.sync_copy(buf, out_hbm.at[pl.ds((base + r) * 128, 128)])

        @pl.loop(0, r_per_w, step=_IDXR)
        def _(r0):
            pltpu.sync_copy(v_hbm.at[pl.ds(base + r0, _IDXR)], vall)
            gather(rows_a, sem_a, 0).start()

            @pl.loop(0, _IDXR, step=2)
            def _(t):
                gather(rows_b, sem_b, t + 1).start()
                gather(rows_a, sem_a, t).wait()
                put(rows_a, r0 + t)

                @pl.when(t + 2 < _IDXR)
                def _():
                    gather(rows_a, sem_a, t + 2).start()

                gather(rows_b, sem_b, t + 1).wait()
                put(rows_b, r0 + t + 1)

    return k


_NB = 1000
_NGRID = _N // _NB


def _combine_body(srows, xp, ntc, ntt, ett, xf_ref, z_ref):
    cnt = jnp.sum((xp[...] != 0).astype(jnp.float32), axis=1, keepdims=True)
    recip = 1.0 / jnp.maximum(cnt, 1.0)
    sr = srows[...]
    ssum = sr[:, 0 * _D:1 * _D]
    for p in range(1, 5):
        ssum = ssum + sr[:, p * _D:(p + 1) * _D]
    sub = ssum * recip
    oh = (lax.broadcasted_iota(jnp.int32, (_NB, 32), 1) == ntc[...]).astype(
        jnp.float32)
    xf = sub + jnp.dot(oh, ntt[...], preferred_element_type=jnp.float32,
                       precision=lax.Precision.HIGHEST)
    xf_ref[...] = xf
    for t in range(_ET):
        z_ref[t] = jnp.maximum(xf + ett[t], 0.0)


_combine = pl.pallas_call(
    _combine_body,
    grid=(_NGRID,),
    in_specs=[
        pl.BlockSpec((_NB, 5 * _D), lambda i: (i, 0)),
        pl.BlockSpec((_NB, 8), lambda i: (i, 0)),
        pl.BlockSpec((_NB, 1), lambda i: (i, 0)),
        pl.BlockSpec((32, _D), lambda i: (0, 0)),
        pl.BlockSpec((16, _D), lambda i: (0, 0)),
    ],
    out_specs=[
        pl.BlockSpec((_NB, _D), lambda i: (i, 0)),
        pl.BlockSpec((_ET, _NB, _D), lambda i: (0, i, 0)),
    ],
    out_shape=[
        jax.ShapeDtypeStruct((_N, _D), jnp.float32),
        jax.ShapeDtypeStruct((_ET, _N, _D), jnp.float32),
    ],
)


def _mlp1_body(xf, a0, a1, w1, b1, h_ref, s1_ref, s2_ref):
    i = pl.program_id(0)
    h0 = xf[...] + a0[...] + a1[...]
    h = jnp.dot(h0, w1[...], preferred_element_type=jnp.float32) + b1[...]
    h_ref[...] = h
    ps = jnp.sum(h, axis=0, keepdims=True)
    pq = jnp.sum(h * h, axis=0, keepdims=True)

    @pl.when(i == 0)
    def _():
        s1_ref[...] = ps
        s2_ref[...] = pq

    @pl.when(i > 0)
    def _():
        s1_ref[...] += ps
        s2_ref[...] += pq


_mlp1 = pl.pallas_call(
    _mlp1_body,
    grid=(_NGRID,),
    in_specs=[
        pl.BlockSpec((_NB, _D), lambda i: (i, 0)),
        pl.BlockSpec((_NB, _D), lambda i: (i, 0)),
        pl.BlockSpec((_NB, _D), lambda i: (i, 0)),
        pl.BlockSpec((_D, 2 * _D), lambda i: (0, 0)),
        pl.BlockSpec((1, 2 * _D), lambda i: (0, 0)),
    ],
    out_specs=[
        pl.BlockSpec((_NB, 2 * _D), lambda i: (i, 0)),
        pl.BlockSpec((1, 2 * _D), lambda i: (0, 0)),
        pl.BlockSpec((1, 2 * _D), lambda i: (0, 0)),
    ],
    out_shape=[
        jax.ShapeDtypeStruct((_N, 2 * _D), jnp.float32),
        jax.ShapeDtypeStruct((1, 2 * _D), jnp.float32),
        jax.ShapeDtypeStruct((1, 2 * _D), jnp.float32),
    ],
)


def _mlp2_core(h, s1, s2, g, be, w2, b2):
    mu = s1[...] * (1.0 / _N)
    var = s2[...] * (1.0 / _N) - mu * mu
    inv = lax.rsqrt(var + 1e-5)
    hn = (h[...] - mu) * (inv * g[...]) + be[...]
    r = jnp.maximum(hn, 0.0)
    return jnp.dot(r, w2[...], preferred_element_type=jnp.float32) + b2[...]


def _mlp2z_body(h, s1, s2, g, be, w2, b2, ett, xf_ref, z_ref):
    o = _mlp2_core(h, s1, s2, g, be, w2, b2)
    xf_ref[...] = o
    for t in range(_ET):
        z_ref[t] = jnp.maximum(o + ett[t], 0.0)


def _mlp2_body(h, s1, s2, g, be, w2, b2, xf_ref):
    xf_ref[...] = _mlp2_core(h, s1, s2, g, be, w2, b2)


_mlp2_common_specs = [
    pl.BlockSpec((_NB, 2 * _D), lambda i: (i, 0)),
    pl.BlockSpec((1, 2 * _D), lambda i: (0, 0)),
    pl.BlockSpec((1, 2 * _D), lambda i: (0, 0)),
    pl.BlockSpec((1, 2 * _D), lambda i: (0, 0)),
    pl.BlockSpec((1, 2 * _D), lambda i: (0, 0)),
    pl.BlockSpec((2 * _D, _D), lambda i: (0, 0)),
    pl.BlockSpec((1, _D), lambda i: (0, 0)),
]

_mlp2z = pl.pallas_call(
    _mlp2z_body,
    grid=(_NGRID,),
    in_specs=_mlp2_common_specs + [pl.BlockSpec((16, _D), lambda i: (0, 0))],
    out_specs=[
        pl.BlockSpec((_NB, _D), lambda i: (i, 0)),
        pl.BlockSpec((_ET, _NB, _D), lambda i: (0, i, 0)),
    ],
    out_shape=[
        jax.ShapeDtypeStruct((_N, _D), jnp.float32),
        jax.ShapeDtypeStruct((_ET, _N, _D), jnp.float32),
    ],
)

_mlp2 = pl.pallas_call(
    _mlp2_body,
    grid=(_NGRID,),
    in_specs=_mlp2_common_specs,
    out_specs=[pl.BlockSpec((_NB, _D), lambda i: (i, 0))],
    out_shape=[jax.ShapeDtypeStruct((_N, _D), jnp.float32)],
)


def _pad_ids(ids, total, fill=None):
    ids = ids.astype(jnp.int32)
    npad = total - ids.shape[0]
    if fill is None:
        pad = jnp.arange(npad, dtype=jnp.int32) % _N
    else:
        pad = _N + jnp.arange(npad, dtype=jnp.int32) % (_NACC - _N)
    return jnp.concatenate([ids, pad]).reshape(total // 128, 128)


def kernel(x, node_type, edge_type, edge_index, st_table, nt_table, et_table,
           W1_0, b1_0, g_0, be_0, W2_0, b2_0,
           W1_1, b1_1, g_1, be_1, W2_1, b2_1,
           W1_2, b1_2, g_2, be_2, W2_2, b2_2):
    src = edge_index[0].astype(jnp.int32)
    dst = edge_index[1].astype(jnp.int32)
    et = edge_type.astype(jnp.int32)

    e_pad = _NW * _EPW_MSG
    v2 = _pad_ids(et * _N + src, e_pad)
    d2 = _pad_ids(dst, e_pad, 'trash')

    n_ids = _NW * _EPW_EMB
    xe = _pad_ids(x.reshape(-1), n_ids)

    zero_blk = jnp.zeros((_ZCH, _D), jnp.float32)
    st_z = st_table.at[0].set(0.0)

    srows = _make_sc_gather(_EPW_EMB, n_ids)(st_z, xe)
    srows = srows[:_N * x.shape[1]].reshape(_N, x.shape[1] * _D)

    xpad = jnp.pad(x.astype(jnp.int32), ((0, 0), (0, 8 - x.shape[1])))
    ntc = node_type.astype(jnp.int32).reshape(_N, 1)
    ntt = jnp.pad(nt_table, ((0, 32 - _NT), (0, 0)))
    ett = jnp.pad(et_table, ((0, 16 - _ET), (0, 0)))
    xf, z = _combine(srows, xpad, ntc, ntt, ett)

    layers = [
        (W1_0, b1_0, g_0, be_0, W2_0, b2_0),
        (W1_1, b1_1, g_1, be_1, W2_1, b2_1),
        (W1_2, b1_2, g_2, be_2, W2_2, b2_2),
    ]
    for l, (w1, b1, g, be, w2, b2) in enumerate(layers):
        aggr = _make_sc_gather_scatter_add(_EPW_MSG)(
            z.reshape(_ET * _N, _D), v2, d2, zero_blk)
        h, s1, s2 = _mlp1(xf, aggr[:_N], aggr[_NOUT:_NOUT + _N], w1,
                          b1.reshape(1, -1))
        args = (h, s1, s2, g.reshape(1, -1), be.reshape(1, -1), w2,
                b2.reshape(1, -1))
        if l < 2:
            xf, z = _mlp2z(*args, ett)
        else:
            (xf,) = _mlp2(*args)
    return xf


# speedup vs baseline: 4.0672x; 1.0227x over previous
import functools

import jax
import jax.numpy as jnp
from jax import lax
from jax.experimental import pallas as pl
from jax.experimental.pallas import tpu as pltpu
from jax.experimental.pallas import tpu_sc as plsc

_N = 10000
_E = 320000
_D = 128
_NT = 17
_ET = 9

_NC = 2
_NS = 16
_NW = _NC * _NS

_NACC = 10240
_ZCH = 640
_WCH = 632
_NOUT = _NS * _WCH


_EPW_MSG = 10240
_EPW_EMB = 2048
_IDXR = 16


@functools.lru_cache(maxsize=None)
def _make_sc_gather_scatter_add(rows_per_worker):
    r_per_w = rows_per_worker // 128

    mesh = plsc.VectorSubcoreMesh(core_axis_name="c", subcore_axis_name="s")

    n_blocks = r_per_w // _IDXR

    @functools.partial(
        pl.kernel,
        out_type=jax.ShapeDtypeStruct((2 * _NOUT, _D), jnp.float32),
        mesh=mesh,
        scratch_types=[
            pltpu.VMEM((2, _IDXR, 128), jnp.int32),
            pltpu.VMEM((2, _IDXR, 128), jnp.int32),
            pltpu.VMEM((128, _D), jnp.float32),
            pltpu.VMEM((128, _D), jnp.float32),
            pltpu.VMEM_SHARED((_NACC, _D), jnp.float32),
            pltpu.SemaphoreType.DMA,
            pltpu.SemaphoreType.DMA,
            pltpu.SemaphoreType.DMA,
            pltpu.SemaphoreType.DMA,
        ],
    )
    def k(tab_hbm, v_hbm, d_hbm, zero_hbm, out_hbm, vall, dall, rows_a, rows_b,
          accum, sem_a, sem_b, sem_i0, sem_i1):
        c = lax.axis_index("c")
        s = lax.axis_index("s")
        wid = c * _NS + s
        base = wid * r_per_w
        isems = (sem_i0, sem_i1)

        def idx_pair(par, blk):
            r0 = base + blk * _IDXR
            return (pltpu.make_async_copy(v_hbm.at[pl.ds(r0, _IDXR)],
                                          vall.at[par], isems[par]),
                    pltpu.make_async_copy(d_hbm.at[pl.ds(r0, _IDXR)],
                                          dall.at[par], isems[par]))

        for cp in idx_pair(0, 0):
            cp.start()

        pltpu.sync_copy(zero_hbm, accum.at[pl.ds(s * _ZCH, _ZCH)])
        plsc.subcore_barrier()

        def gather(buf, sem, par, j):
            return pltpu.make_async_copy(tab_hbm.at[vall.at[par, j]], buf, sem)

        for b in range(n_blocks):
            par = b % 2
            for cp in idx_pair(par, b):
                cp.wait()
            if b + 1 < n_blocks:
                for cp in idx_pair(1 - par, b + 1):
                    cp.start()

            gather(rows_a, sem_a, par, 0).start()

            @pl.loop(0, _IDXR, step=2)
            def _(t):
                gather(rows_b, sem_b, par, t + 1).start()
                gather(rows_a, sem_a, par, t).wait()
                pltpu.sync_copy(rows_a, accum.at[dall.at[par, t]], add=True)

                @pl.when(t + 2 < _IDXR)
                def _():
                    gather(rows_a, sem_a, par, t + 2).start()

                gather(rows_b, sem_b, par, t + 1).wait()
                pltpu.sync_copy(rows_b, accum.at[dall.at[par, t + 1]],
                                add=True)

        plsc.subcore_barrier()
        pltpu.sync_copy(
            accum.at[pl.ds(s * _WCH, _WCH)],
            out_hbm.at[pl.ds(c * _NOUT + s * _WCH, _WCH)],
        )

    return k


@functools.lru_cache(maxsize=None)
def _make_sc_gather(rows_per_worker, n_out_rows):
    r_per_w = rows_per_worker // 128

    mesh = plsc.VectorSubcoreMesh(core_axis_name="c", subcore_axis_name="s")

    @functools.partial(
        pl.kernel,
        out_type=jax.ShapeDtypeStruct((n_out_rows, _D), jnp.float32),
        mesh=mesh,
        scratch_types=[
            pltpu.VMEM((_IDXR, 128), jnp.int32),
            pltpu.VMEM((128, _D), jnp.float32),
            pltpu.VMEM((128, _D), jnp.float32),
            pltpu.SemaphoreType.DMA,
            pltpu.SemaphoreType.DMA,
        ],
    )
    def k(tab_hbm, v_hbm, out_hbm, vall, rows_a, rows_b, sem_a, sem_b):
        c = lax.axis_index("c")
        s = lax.axis_index("s")
        wid = c * _NS + s
        base = wid * r_per_w

        def gather(buf, sem, j):
            return pltpu.make_async_copy(tab_hbm.at[vall.at[j]], buf, sem)

        def put(buf, r):
            pltpu.sync_copy(buf, out_hbm.at[pl.ds((base + r) * 128, 128)])

        @pl.loop(0, r_per_w, step=_IDXR)
        def _(r0):
            pltpu.sync_copy(v_hbm.at[pl.ds(base + r0, _IDXR)], vall)
            gather(rows_a, sem_a, 0).start()

            @pl.loop(0, _IDXR, step=2)
            def _(t):
                gather(rows_b, sem_b, t + 1).start()
                gather(rows_a, sem_a, t).wait()
                put(rows_a, r0 + t)

                @pl.when(t + 2 < _IDXR)
                def _():
                    gather(rows_a, sem_a, t + 2).start()

                gather(rows_b, sem_b, t + 1).wait()
                put(rows_b, r0 + t + 1)

    return k


_NB = 1000
_NGRID = _N // _NB


def _combine_body(srows, xp, ntc, ntt, ett, xf_ref, z_ref):
    cnt = jnp.sum((xp[...] != 0).astype(jnp.float32), axis=1, keepdims=True)
    recip = 1.0 / jnp.maximum(cnt, 1.0)
    sr = srows[...]
    ssum = sr[:, 0 * _D:1 * _D]
    for p in range(1, 5):
        ssum = ssum + sr[:, p * _D:(p + 1) * _D]
    sub = ssum * recip
    oh = (lax.broadcasted_iota(jnp.int32, (_NB, 32), 1) == ntc[...]).astype(
        jnp.float32)
    xf = sub + jnp.dot(oh, ntt[...], preferred_element_type=jnp.float32,
                       precision=lax.Precision.HIGHEST)
    xf_ref[...] = xf
    for t in range(_ET):
        z_ref[t] = jnp.maximum(xf + ett[t], 0.0)


_combine = pl.pallas_call(
    _combine_body,
    grid=(_NGRID,),
    in_specs=[
        pl.BlockSpec((_NB, 5 * _D), lambda i: (i, 0)),
        pl.BlockSpec((_NB, 8), lambda i: (i, 0)),
        pl.BlockSpec((_NB, 1), lambda i: (i, 0)),
        pl.BlockSpec((32, _D), lambda i: (0, 0)),
        pl.BlockSpec((16, _D), lambda i: (0, 0)),
    ],
    out_specs=[
        pl.BlockSpec((_NB, _D), lambda i: (i, 0)),
        pl.BlockSpec((_ET, _NB, _D), lambda i: (0, i, 0)),
    ],
    out_shape=[
        jax.ShapeDtypeStruct((_N, _D), jnp.float32),
        jax.ShapeDtypeStruct((_ET, _N, _D), jnp.float32),
    ],
)


def _mlp1_body(xf, a0, a1, w1, b1, h_ref, s1_ref, s2_ref):
    i = pl.program_id(0)
    h0 = xf[...] + a0[...] + a1[...]
    h = jnp.dot(h0, w1[...], preferred_element_type=jnp.float32) + b1[...]
    h_ref[...] = h
    ps = jnp.sum(h, axis=0, keepdims=True)
    pq = jnp.sum(h * h, axis=0, keepdims=True)

    @pl.when(i == 0)
    def _():
        s1_ref[...] = ps
        s2_ref[...] = pq

    @pl.when(i > 0)
    def _():
        s1_ref[...] += ps
        s2_ref[...] += pq


_mlp1 = pl.pallas_call(
    _mlp1_body,
    grid=(_NGRID,),
    in_specs=[
        pl.BlockSpec((_NB, _D), lambda i: (i, 0)),
        pl.BlockSpec((_NB, _D), lambda i: (i, 0)),
        pl.BlockSpec((_NB, _D), lambda i: (i, 0)),
        pl.BlockSpec((_D, 2 * _D), lambda i: (0, 0)),
        pl.BlockSpec((1, 2 * _D), lambda i: (0, 0)),
    ],
    out_specs=[
        pl.BlockSpec((_NB, 2 * _D), lambda i: (i, 0)),
        pl.BlockSpec((1, 2 * _D), lambda i: (0, 0)),
        pl.BlockSpec((1, 2 * _D), lambda i: (0, 0)),
    ],
    out_shape=[
        jax.ShapeDtypeStruct((_N, 2 * _D), jnp.float32),
        jax.ShapeDtypeStruct((1, 2 * _D), jnp.float32),
        jax.ShapeDtypeStruct((1, 2 * _D), jnp.float32),
    ],
)


def _mlp2_core(h, s1, s2, g, be, w2, b2):
    mu = s1[...] * (1.0 / _N)
    var = s2[...] * (1.0 / _N) - mu * mu
    inv = lax.rsqrt(var + 1e-5)
    hn = (h[...] - mu) * (inv * g[...]) + be[...]
    r = jnp.maximum(hn, 0.0)
    return jnp.dot(r, w2[...], preferred_element_type=jnp.float32) + b2[...]


def _mlp2z_body(h, s1, s2, g, be, w2, b2, ett, xf_ref, z_ref):
    o = _mlp2_core(h, s1, s2, g, be, w2, b2)
    xf_ref[...] = o
    for t in range(_ET):
        z_ref[t] = jnp.maximum(o + ett[t], 0.0)


def _mlp2_body(h, s1, s2, g, be, w2, b2, xf_ref):
    xf_ref[...] = _mlp2_core(h, s1, s2, g, be, w2, b2)


_mlp2_common_specs = [
    pl.BlockSpec((_NB, 2 * _D), lambda i: (i, 0)),
    pl.BlockSpec((1, 2 * _D), lambda i: (0, 0)),
    pl.BlockSpec((1, 2 * _D), lambda i: (0, 0)),
    pl.BlockSpec((1, 2 * _D), lambda i: (0, 0)),
    pl.BlockSpec((1, 2 * _D), lambda i: (0, 0)),
    pl.BlockSpec((2 * _D, _D), lambda i: (0, 0)),
    pl.BlockSpec((1, _D), lambda i: (0, 0)),
]

_mlp2z = pl.pallas_call(
    _mlp2z_body,
    grid=(_NGRID,),
    in_specs=_mlp2_common_specs + [pl.BlockSpec((16, _D), lambda i: (0, 0))],
    out_specs=[
        pl.BlockSpec((_NB, _D), lambda i: (i, 0)),
        pl.BlockSpec((_ET, _NB, _D), lambda i: (0, i, 0)),
    ],
    out_shape=[
        jax.ShapeDtypeStruct((_N, _D), jnp.float32),
        jax.ShapeDtypeStruct((_ET, _N, _D), jnp.float32),
    ],
)

_mlp2 = pl.pallas_call(
    _mlp2_body,
    grid=(_NGRID,),
    in_specs=_mlp2_common_specs,
    out_specs=[pl.BlockSpec((_NB, _D), lambda i: (i, 0))],
    out_shape=[jax.ShapeDtypeStruct((_N, _D), jnp.float32)],
)


def _pad_ids(ids, total, fill=None):
    ids = ids.astype(jnp.int32)
    npad = total - ids.shape[0]
    if fill is None:
        pad = jnp.arange(npad, dtype=jnp.int32) % _N
    else:
        pad = _N + jnp.arange(npad, dtype=jnp.int32) % (_NACC - _N)
    return jnp.concatenate([ids, pad]).reshape(total // 128, 128)


def kernel(x, node_type, edge_type, edge_index, st_table, nt_table, et_table,
           W1_0, b1_0, g_0, be_0, W2_0, b2_0,
           W1_1, b1_1, g_1, be_1, W2_1, b2_1,
           W1_2, b1_2, g_2, be_2, W2_2, b2_2):
    src = edge_index[0].astype(jnp.int32)
    dst = edge_index[1].astype(jnp.int32)
    et = edge_type.astype(jnp.int32)

    e_pad = _NW * _EPW_MSG
    v2 = _pad_ids(et * _N + src, e_pad)
    d2 = _pad_ids(dst, e_pad, 'trash')

    n_ids = _NW * _EPW_EMB
    xe = _pad_ids(x.reshape(-1), n_ids)

    zero_blk = jnp.zeros((_ZCH, _D), jnp.float32)
    st_z = st_table.at[0].set(0.0)

    srows = _make_sc_gather(_EPW_EMB, n_ids)(st_z, xe)
    srows = srows[:_N * x.shape[1]].reshape(_N, x.shape[1] * _D)

    xpad = jnp.pad(x.astype(jnp.int32), ((0, 0), (0, 8 - x.shape[1])))
    ntc = node_type.astype(jnp.int32).reshape(_N, 1)
    ntt = jnp.pad(nt_table, ((0, 32 - _NT), (0, 0)))
    ett = jnp.pad(et_table, ((0, 16 - _ET), (0, 0)))
    xf, z = _combine(srows, xpad, ntc, ntt, ett)

    layers = [
        (W1_0, b1_0, g_0, be_0, W2_0, b2_0),
        (W1_1, b1_1, g_1, be_1, W2_1, b2_1),
        (W1_2, b1_2, g_2, be_2, W2_2, b2_2),
    ]
    for l, (w1, b1, g, be, w2, b2) in enumerate(layers):
        aggr = _make_sc_gather_scatter_add(_EPW_MSG)(
            z.reshape(_ET * _N, _D), v2, d2, zero_blk)
        h, s1, s2 = _mlp1(xf, aggr[:_N], aggr[_NOUT:_NOUT + _N], w1,
                          b1.reshape(1, -1))
        args = (h, s1, s2, g.reshape(1, -1), be.reshape(1, -1), w2,
                b2.reshape(1, -1))
        if l < 2:
            xf, z = _mlp2z(*args, ett)
        else:
            (xf,) = _mlp2(*args)
    return xf


# speedup vs baseline: 4.2337x; 1.0409x over previous
import functools

import jax
import jax.numpy as jnp
from jax import lax
from jax.experimental import pallas as pl
from jax.experimental.pallas import tpu as pltpu
from jax.experimental.pallas import tpu_sc as plsc

_N = 10000
_E = 320000
_D = 128
_NT = 17
_ET = 9

_NC = 2
_NS = 16
_NW = _NC * _NS

_NACC = 10240
_ZCH = 640
_WCH = 632
_NOUT = _NS * _WCH


_EPW_MSG = 10240
_EPW_EMB = 2048
_IDXR = 16


@functools.lru_cache(maxsize=None)
def _make_sc_gather_scatter_add(rows_per_worker):
    r_per_w = rows_per_worker // 128

    mesh = plsc.VectorSubcoreMesh(core_axis_name="c", subcore_axis_name="s")

    n_blocks = r_per_w // _IDXR

    @functools.partial(
        pl.kernel,
        out_type=jax.ShapeDtypeStruct((2 * _NOUT, _D), jnp.float32),
        mesh=mesh,
        scratch_types=[
            pltpu.VMEM((2, _IDXR, 128), jnp.int32),
            pltpu.VMEM((2, _IDXR, 128), jnp.int32),
            pltpu.VMEM((128, _D), jnp.float32),
            pltpu.VMEM((128, _D), jnp.float32),
            pltpu.VMEM_SHARED((_NACC, _D), jnp.float32),
            pltpu.SemaphoreType.DMA,
            pltpu.SemaphoreType.DMA,
            pltpu.SemaphoreType.DMA,
            pltpu.SemaphoreType.DMA,
        ],
    )
    def k(tab_hbm, v_hbm, d_hbm, zero_hbm, out_hbm, vall, dall, rows_a, rows_b,
          accum, sem_a, sem_b, sem_i0, sem_i1):
        c = lax.axis_index("c")
        s = lax.axis_index("s")
        wid = c * _NS + s
        base = wid * r_per_w
        isems = (sem_i0, sem_i1)

        def idx_pair(par, blk):
            r0 = base + blk * _IDXR
            return (pltpu.make_async_copy(v_hbm.at[pl.ds(r0, _IDXR)],
                                          vall.at[par], isems[par]),
                    pltpu.make_async_copy(d_hbm.at[pl.ds(r0, _IDXR)],
                                          dall.at[par], isems[par]))

        for cp in idx_pair(0, 0):
            cp.start()

        pltpu.sync_copy(zero_hbm, accum.at[pl.ds(s * _ZCH, _ZCH)])
        plsc.subcore_barrier()

        def gather(buf, sem, par, j):
            return pltpu.make_async_copy(tab_hbm.at[vall.at[par, j]], buf, sem)

        for b in range(n_blocks):
            par = b % 2
            for cp in idx_pair(par, b):
                cp.wait()
            if b + 1 < n_blocks:
                for cp in idx_pair(1 - par, b + 1):
                    cp.start()

            gather(rows_a, sem_a, par, 0).start()

            @pl.loop(0, _IDXR, step=2)
            def _(t):
                gather(rows_b, sem_b, par, t + 1).start()
                gather(rows_a, sem_a, par, t).wait()
                pltpu.sync_copy(rows_a, accum.at[dall.at[par, t]], add=True)

                @pl.when(t + 2 < _IDXR)
                def _():
                    gather(rows_a, sem_a, par, t + 2).start()

                gather(rows_b, sem_b, par, t + 1).wait()
                pltpu.sync_copy(rows_b, accum.at[dall.at[par, t + 1]],
                                add=True)

        plsc.subcore_barrier()
        pltpu.sync_copy(
            accum.at[pl.ds(s * _WCH, _WCH)],
            out_hbm.at[pl.ds(c * _NOUT + s * _WCH, _WCH)],
        )

    return k


@functools.lru_cache(maxsize=None)
def _make_sc_gather(rows_per_worker, n_out_rows):
    r_per_w = rows_per_worker // 128

    mesh = plsc.VectorSubcoreMesh(core_axis_name="c", subcore_axis_name="s")

    @functools.partial(
        pl.kernel,
        out_type=jax.ShapeDtypeStruct((n_out_rows, _D), jnp.float32),
        mesh=mesh,
        scratch_types=[
            pltpu.VMEM((_IDXR, 128), jnp.int32),
            pltpu.VMEM((128, _D), jnp.float32),
            pltpu.VMEM((128, _D), jnp.float32),
            pltpu.SemaphoreType.DMA,
            pltpu.SemaphoreType.DMA,
        ],
    )
    def k(tab_hbm, v_hbm, out_hbm, vall, rows_a, rows_b, sem_a, sem_b):
        c = lax.axis_index("c")
        s = lax.axis_index("s")
        wid = c * _NS + s
        base = wid * r_per_w

        def gather(buf, sem, j):
            return pltpu.make_async_copy(tab_hbm.at[vall.at[j]], buf, sem)

        def put(buf, r):
            pltpu.sync_copy(buf, out_hbm.at[pl.ds((base + r) * 128, 128)])

        @pl.loop(0, r_per_w, step=_IDXR)
        def _(r0):
            pltpu.sync_copy(v_hbm.at[pl.ds(base + r0, _IDXR)], vall)
            gather(rows_a, sem_a, 0).start()

            @pl.loop(0, _IDXR, step=2)
            def _(t):
                gather(rows_b, sem_b, t + 1).start()
                gather(rows_a, sem_a, t).wait()
                put(rows_a, r0 + t)

                @pl.when(t + 2 < _IDXR)
                def _():
                    gather(rows_a, sem_a, t + 2).start()

                gather(rows_b, sem_b, t + 1).wait()
                put(rows_b, r0 + t + 1)

    return k


_NB = 1000
_NGRID = _N // _NB


def _combine_body(srows, xp, ntc, ntt, ett, xf_ref, z_ref):
    cnt = jnp.sum((xp[...] != 0).astype(jnp.float32), axis=1, keepdims=True)
    recip = 1.0 / jnp.maximum(cnt, 1.0)
    sr = srows[...]
    ssum = sr[:, 0 * _D:1 * _D]
    for p in range(1, 5):
        ssum = ssum + sr[:, p * _D:(p + 1) * _D]
    sub = ssum * recip
    oh = (lax.broadcasted_iota(jnp.int32, (_NB, 32), 1) == ntc[...]).astype(
        jnp.float32)
    xf = sub + jnp.dot(oh, ntt[...], preferred_element_type=jnp.float32,
                       precision=lax.Precision.HIGHEST)
    xf_ref[...] = xf
    for t in range(_ET):
        z_ref[t] = jnp.maximum(xf + ett[t], 0.0)


_combine = pl.pallas_call(
    _combine_body,
    grid=(_NGRID,),
    in_specs=[
        pl.BlockSpec((_NB, 5 * _D), lambda i: (i, 0)),
        pl.BlockSpec((_NB, 8), lambda i: (i, 0)),
        pl.BlockSpec((_NB, 1), lambda i: (i, 0)),
        pl.BlockSpec((32, _D), lambda i: (0, 0)),
        pl.BlockSpec((16, _D), lambda i: (0, 0)),
    ],
    out_specs=[
        pl.BlockSpec((_NB, _D), lambda i: (i, 0)),
        pl.BlockSpec((_ET, _NB, _D), lambda i: (0, i, 0)),
    ],
    out_shape=[
        jax.ShapeDtypeStruct((_N, _D), jnp.float32),
        jax.ShapeDtypeStruct((_ET, _N, _D), jnp.float32),
    ],
)


def _mlp_core(h, s1_ref, s2_ref, g, be, w2, b2):
    mu = s1_ref[...] * (1.0 / _N)
    var = s2_ref[...] * (1.0 / _N) - mu * mu
    inv = lax.rsqrt(var + 1e-5)
    hn = (h - mu) * (inv * g[...]) + be[...]
    r = jnp.maximum(hn, 0.0)
    return jnp.dot(r, w2[...], preferred_element_type=jnp.float32) + b2[...]


def _make_mlp(build_z):
    def body(xf, a0, a1, w1, b1, g, be, w2, b2, ett, xf_ref, *refs):
        ph = pl.program_id(0)
        i = pl.program_id(1)
        if build_z:
            z_ref, h_scr, s1_ref, s2_ref = refs
        else:
            h_scr, s1_ref, s2_ref = refs

        @pl.when(ph == 0)
        def _():
            h0 = xf[...] + a0[...] + a1[...]
            h = jnp.dot(h0, w1[...], preferred_element_type=jnp.float32) \
                + b1[...]
            h_scr[pl.ds(i * _NB, _NB), :] = h
            ps = jnp.sum(h, axis=0, keepdims=True)
            pq = jnp.sum(h * h, axis=0, keepdims=True)

            @pl.when(i == 0)
            def _():
                s1_ref[...] = ps
                s2_ref[...] = pq

            @pl.when(i > 0)
            def _():
                s1_ref[...] += ps
                s2_ref[...] += pq

        @pl.when(ph == 1)
        def _():
            h = h_scr[pl.ds(i * _NB, _NB), :]
            o = _mlp_core(h, s1_ref, s2_ref, g, be, w2, b2)
            xf_ref[...] = o
            if build_z:
                for t in range(_ET):
                    z_ref[t] = jnp.maximum(o + ett[t], 0.0)

    in_specs = [
        pl.BlockSpec((_NB, _D), lambda ph, i: (i * (1 - ph), 0)),
        pl.BlockSpec((_NB, _D), lambda ph, i: (i * (1 - ph), 0)),
        pl.BlockSpec((_NB, _D), lambda ph, i: (i * (1 - ph), 0)),
        pl.BlockSpec((_D, 2 * _D), lambda ph, i: (0, 0)),
        pl.BlockSpec((1, 2 * _D), lambda ph, i: (0, 0)),
        pl.BlockSpec((1, 2 * _D), lambda ph, i: (0, 0)),
        pl.BlockSpec((1, 2 * _D), lambda ph, i: (0, 0)),
        pl.BlockSpec((2 * _D, _D), lambda ph, i: (0, 0)),
        pl.BlockSpec((1, _D), lambda ph, i: (0, 0)),
        pl.BlockSpec((16, _D), lambda ph, i: (0, 0)),
    ]
    out_specs = [pl.BlockSpec((_NB, _D), lambda ph, i: (i * ph, 0))]
    out_shape = [jax.ShapeDtypeStruct((_N, _D), jnp.float32)]
    if build_z:
        out_specs.append(
            pl.BlockSpec((_ET, _NB, _D), lambda ph, i: (0, i * ph, 0)))
        out_shape.append(jax.ShapeDtypeStruct((_ET, _N, _D), jnp.float32))

    return pl.pallas_call(
        body,
        grid=(2, _NGRID),
        in_specs=in_specs,
        out_specs=out_specs,
        out_shape=out_shape,
        scratch_shapes=[
            pltpu.VMEM((_N, 2 * _D), jnp.float32),
            pltpu.VMEM((1, 2 * _D), jnp.float32),
            pltpu.VMEM((1, 2 * _D), jnp.float32),
        ],
    )


_mlpz = _make_mlp(True)
_mlp = _make_mlp(False)


def _pad_ids(ids, total, fill=None):
    ids = ids.astype(jnp.int32)
    npad = total - ids.shape[0]
    if fill is None:
        pad = jnp.arange(npad, dtype=jnp.int32) % _N
    else:
        pad = _N + jnp.arange(npad, dtype=jnp.int32) % (_NACC - _N)
    return jnp.concatenate([ids, pad]).reshape(total // 128, 128)


def kernel(x, node_type, edge_type, edge_index, st_table, nt_table, et_table,
           W1_0, b1_0, g_0, be_0, W2_0, b2_0,
           W1_1, b1_1, g_1, be_1, W2_1, b2_1,
           W1_2, b1_2, g_2, be_2, W2_2, b2_2):
    src = edge_index[0].astype(jnp.int32)
    dst = edge_index[1].astype(jnp.int32)
    et = edge_type.astype(jnp.int32)

    e_pad = _NW * _EPW_MSG
    v2 = _pad_ids(et * _N + src, e_pad)
    d2 = _pad_ids(dst, e_pad, 'trash')

    n_ids = _NW * _EPW_EMB
    xe = _pad_ids(x.reshape(-1), n_ids)

    zero_blk = jnp.zeros((_ZCH, _D), jnp.float32)
    st_z = st_table.at[0].set(0.0)

    srows = _make_sc_gather(_EPW_EMB, n_ids)(st_z, xe)
    srows = srows[:_N * x.shape[1]].reshape(_N, x.shape[1] * _D)

    xpad = jnp.pad(x.astype(jnp.int32), ((0, 0), (0, 8 - x.shape[1])))
    ntc = node_type.astype(jnp.int32).reshape(_N, 1)
    ntt = jnp.pad(nt_table, ((0, 32 - _NT), (0, 0)))
    ett = jnp.pad(et_table, ((0, 16 - _ET), (0, 0)))
    xf, z = _combine(srows, xpad, ntc, ntt, ett)

    layers = [
        (W1_0, b1_0, g_0, be_0, W2_0, b2_0),
        (W1_1, b1_1, g_1, be_1, W2_1, b2_1),
        (W1_2, b1_2, g_2, be_2, W2_2, b2_2),
    ]
    for l, (w1, b1, g, be, w2, b2) in enumerate(layers):
        aggr = _make_sc_gather_scatter_add(_EPW_MSG)(
            z.reshape(_ET * _N, _D), v2, d2, zero_blk)
        args = (xf, aggr[:_N], aggr[_NOUT:_NOUT + _N], w1, b1.reshape(1, -1),
                g.reshape(1, -1), be.reshape(1, -1), w2, b2.reshape(1, -1),
                ett)
        if l < 2:
            xf, z = _mlpz(*args)
        else:
            (xf,) = _mlp(*args)
    return xf
